# baseline (speedup 1.0000x reference)
import functools

import jax
import jax.numpy as jnp
from jax import lax
from jax.experimental import pallas as pl
from jax.experimental.pallas import tpu as pltpu
from jax.experimental.pallas import tpu_sc as plsc

BS = 1024
TSTPS = 50
N_NODES = 26
OUT_DIM = 64
TYPE_DIM = 3
TPAD = 8
ROWS = BS * TSTPS
PAIRW = 2 * OUT_DIM

_CHUNK = 128


def _sc_node_gather(node4, gid):
    info = plsc.get_sparse_core_info()
    nw = info.num_cores * info.num_subcores
    b_per_w = BS // nw

    mesh = plsc.VectorSubcoreMesh(core_axis_name="c", subcore_axis_name="s")

    @functools.partial(
        pl.kernel,
        mesh=mesh,
        out_type=jax.ShapeDtypeStruct((BS, TSTPS, OUT_DIM), jnp.float32),
        scratch_types=[
            pltpu.VMEM((b_per_w,), jnp.int32),
            pltpu.SemaphoreType.DMA,
        ],
    )
    def k(node_hbm, gid_hbm, ng_out, gidx_v, sem):
        wid = lax.axis_index("s") * info.num_cores + lax.axis_index("c")
        base = wid * b_per_w
        pltpu.sync_copy(gid_hbm.at[wid], gidx_v)
        copies = []
        for i in range(b_per_w):
            if i % 16 == 0:
                gvec = gidx_v[pl.ds(i, 16)]
            g = gvec[i % 16]
            copies.append(pltpu.async_copy(
                node_hbm.at[base + i, :, g, :], ng_out.at[base + i], sem))
        for c in copies:
            c.wait()

    return k(node4, gid)


def _sc_type_gather(table, gid):
    info = plsc.get_sparse_core_info()
    nw = info.num_cores * info.num_subcores
    g_per_w = BS // nw

    mesh = plsc.VectorSubcoreMesh(core_axis_name="c", subcore_axis_name="s")

    @functools.partial(
        pl.kernel,
        mesh=mesh,
        compiler_params=pltpu.CompilerParams(use_tc_tiling_on_sc=False),
        out_type=jax.ShapeDtypeStruct((nw, g_per_w, TPAD), jnp.float32),
        scratch_types=[
            pltpu.VMEM((g_per_w,), jnp.int32),
            pltpu.VMEM((g_per_w, TPAD), jnp.float32),
            pltpu.SemaphoreType.DMA,
        ],
    )
    def k(tbl_hbm, gid_hbm, gt_out, gidx_v, gt_v, sem):
        wid = lax.axis_index("s") * info.num_cores + lax.axis_index("c")
        pltpu.sync_copy(gid_hbm.at[wid], gidx_v)
        pltpu.async_copy(tbl_hbm.at[gidx_v], gt_v, sem).wait()
        pltpu.sync_copy(gt_v, gt_out.at[wid])

    return k(table, gid)


_BB = 16
_BLK = _BB * TSTPS


_JB = 4


def _tc_body(gnid_ref, node_ref, gt_ref, w1t_ref, w2t_ref, b_ref, out_ref):
    i = pl.program_id(0)
    w2t = w2t_ref[...]
    gtr = jnp.dot(gt_ref[...][:, 0, :], w1t_ref[...],
                  preferred_element_type=jnp.float32)
    bias = gtr + b_ref[...]
    x = node_ref[...]
    iota_n = lax.broadcasted_iota(jnp.int32, (1, N_NODES, 1), 1)
    for j in range(_JB):
        g = gnid_ref[i * _JB + j]
        y = jnp.sum(jnp.where(iota_n == g, x[j], 0.0), axis=1)
        acc = jnp.dot(y, w2t, preferred_element_type=jnp.float32)
        out_ref[j] = acc + bias[j][None, :]


def _tc_fuse(gnid, node4, gt, w1t, w2t, bvec):
    grid_spec = pltpu.PrefetchScalarGridSpec(
        num_scalar_prefetch=1,
        grid=(BS // _JB,),
        in_specs=[
            pl.BlockSpec((_JB, TSTPS, N_NODES, OUT_DIM),
                         lambda i, g: (i, 0, 0, 0)),
            pl.BlockSpec((_JB, 1, TPAD), lambda i, g: (i, 0, 0)),
            pl.BlockSpec((TPAD, OUT_DIM), lambda i, g: (0, 0)),
            pl.BlockSpec((OUT_DIM, OUT_DIM), lambda i, g: (0, 0)),
            pl.BlockSpec((1, OUT_DIM), lambda i, g: (0, 0)),
        ],
        out_specs=pl.BlockSpec((_JB, TSTPS, OUT_DIM), lambda i, g: (i, 0, 0)),
    )
    return pl.pallas_call(
        _tc_body,
        grid_spec=grid_spec,
        out_shape=jax.ShapeDtypeStruct((BS, TSTPS, OUT_DIM), jnp.float32),
    )(gnid, node4, gt, w1t, w2t, bvec)


def kernel(goal_id, goal_classnode_id, goal_node_id, node_repr, goal_type_table, W, b):
    del goal_classnode_id
    info = plsc.get_sparse_core_info()
    nw = info.num_cores * info.num_subcores

    table_pad = jnp.pad(goal_type_table, ((0, 0), (0, TPAD - TYPE_DIM)))
    gt = _sc_type_gather(
        table_pad, goal_id.astype(jnp.int32).reshape(nw, BS // nw)
    ).reshape(BS, 1, TPAD)

    w1t = jnp.pad(W[:, :TYPE_DIM].T, ((0, TPAD - TYPE_DIM), (0, 0)))
    w2t = W[:, TYPE_DIM:].T
    return _tc_fuse(goal_node_id.astype(jnp.int32), node_repr, gt, w1t, w2t,
                    b.reshape(1, OUT_DIM))


# speedup vs baseline: 1.3488x; 1.3488x over previous
import functools

import jax
import jax.numpy as jnp
from jax import lax
from jax.experimental import pallas as pl
from jax.experimental.pallas import tpu as pltpu
from jax.experimental.pallas import tpu_sc as plsc

BS = 1024
TSTPS = 50
N_NODES = 26
OUT_DIM = 64
TYPE_DIM = 3
TPAD = 8
ROWS = BS * TSTPS
PAIRW = 2 * OUT_DIM

_CHUNK = 128


def _sc_node_gather(node4, gid):
    info = plsc.get_sparse_core_info()
    nw = info.num_cores * info.num_subcores
    b_per_w = BS // nw

    mesh = plsc.VectorSubcoreMesh(core_axis_name="c", subcore_axis_name="s")

    @functools.partial(
        pl.kernel,
        mesh=mesh,
        out_type=jax.ShapeDtypeStruct((BS, TSTPS, OUT_DIM), jnp.float32),
        scratch_types=[
            pltpu.VMEM((b_per_w,), jnp.int32),
            pltpu.VMEM((8, TSTPS, OUT_DIM), jnp.float32),
            pltpu.VMEM((8, TSTPS, OUT_DIM), jnp.float32),
            pltpu.SemaphoreType.DMA,
            pltpu.SemaphoreType.DMA,
        ],
    )
    def k(node_hbm, gid_hbm, ng_out, gidx_v, buf0, buf1, sem, wsem):
        wid = lax.axis_index("s") * info.num_cores + lax.axis_index("c")
        base = wid * b_per_w
        pltpu.sync_copy(gid_hbm.at[wid], gidx_v)
        gv0 = gidx_v[pl.ds(0, 16)]
        gv1 = gidx_v[pl.ds(16, 16)]
        bufs = (buf0, buf1)
        wb = [None, None]
        for r in range(b_per_w // 8):
            bi = r % 2
            if wb[bi] is not None:
                wb[bi].wait()
            gs = []
            for i in range(8):
                idx = r * 8 + i
                g = (gv0 if idx < 16 else gv1)[idx % 16]
                gs.append(pltpu.async_copy(
                    node_hbm.at[base + idx, :, g, :], bufs[bi].at[i], sem))
            for c in gs:
                c.wait()
            wb[bi] = pltpu.async_copy(
                bufs[bi], ng_out.at[pl.ds(base + r * 8, 8)], wsem)
        wb[0].wait()
        wb[1].wait()

    return k(node4, gid)


def _sc_type_gather(table, gid):
    info = plsc.get_sparse_core_info()
    nw = info.num_cores * info.num_subcores
    g_per_w = BS // nw

    mesh = plsc.VectorSubcoreMesh(core_axis_name="c", subcore_axis_name="s")

    @functools.partial(
        pl.kernel,
        mesh=mesh,
        compiler_params=pltpu.CompilerParams(use_tc_tiling_on_sc=False),
        out_type=jax.ShapeDtypeStruct((nw, g_per_w, TPAD), jnp.float32),
        scratch_types=[
            pltpu.VMEM((g_per_w,), jnp.int32),
            pltpu.VMEM((g_per_w, TPAD), jnp.float32),
            pltpu.SemaphoreType.DMA,
        ],
    )
    def k(tbl_hbm, gid_hbm, gt_out, gidx_v, gt_v, sem):
        wid = lax.axis_index("s") * info.num_cores + lax.axis_index("c")
        pltpu.sync_copy(gid_hbm.at[wid], gidx_v)
        pltpu.async_copy(tbl_hbm.at[gidx_v], gt_v, sem).wait()
        pltpu.sync_copy(gt_v, gt_out.at[wid])

    return k(table, gid)


_BB = 16
_BLK = _BB * TSTPS


_BB = 32


def _tc_body(ng_ref, gt_ref, w1t_ref, w2t_ref, b_ref, out_ref):
    ng = ng_ref[...]
    gt = gt_ref[...][:, 0, :]
    acc = lax.dot_general(ng, w2t_ref[...], (((2,), (0,)), ((), ())),
                          preferred_element_type=jnp.float32)
    gtr = jnp.dot(gt, w1t_ref[...], preferred_element_type=jnp.float32)
    out_ref[...] = acc + gtr[:, None, :] + b_ref[...][None, :, :]


def _tc_fuse(ng3, gt, w1t, w2t, bvec):
    grid = (BS // _BB,)
    return pl.pallas_call(
        _tc_body,
        grid=grid,
        in_specs=[
            pl.BlockSpec((_BB, TSTPS, OUT_DIM), lambda i: (i, 0, 0)),
            pl.BlockSpec((_BB, 1, TPAD), lambda i: (i, 0, 0)),
            pl.BlockSpec((TPAD, OUT_DIM), lambda i: (0, 0)),
            pl.BlockSpec((OUT_DIM, OUT_DIM), lambda i: (0, 0)),
            pl.BlockSpec((1, OUT_DIM), lambda i: (0, 0)),
        ],
        out_specs=pl.BlockSpec((_BB, TSTPS, OUT_DIM), lambda i: (i, 0, 0)),
        out_shape=jax.ShapeDtypeStruct((BS, TSTPS, OUT_DIM), jnp.float32),
    )(ng3, gt, w1t, w2t, bvec)


def kernel(goal_id, goal_classnode_id, goal_node_id, node_repr, goal_type_table, W, b):
    del goal_classnode_id
    info = plsc.get_sparse_core_info()
    nw = info.num_cores * info.num_subcores

    gnid = goal_node_id.astype(jnp.int32).reshape(nw, BS // nw)
    ng3 = _sc_node_gather(node_repr, gnid)

    table_pad = jnp.pad(goal_type_table, ((0, 0), (0, TPAD - TYPE_DIM)))
    gt = _sc_type_gather(
        table_pad, goal_id.astype(jnp.int32).reshape(nw, BS // nw)
    ).reshape(BS, 1, TPAD)

    w1t = jnp.pad(W[:, :TYPE_DIM].T, ((0, TPAD - TYPE_DIM), (0, 0)))
    w2t = W[:, TYPE_DIM:].T
    return _tc_fuse(ng3, gt, w1t, w2t, b.reshape(1, OUT_DIM))


# speedup vs baseline: 1.3492x; 1.0003x over previous
import functools

import jax
import jax.numpy as jnp
from jax import lax
from jax.experimental import pallas as pl
from jax.experimental.pallas import tpu as pltpu
from jax.experimental.pallas import tpu_sc as plsc

BS = 1024
TSTPS = 50
N_NODES = 26
OUT_DIM = 64
TYPE_DIM = 3
TPAD = 8
ROWS = BS * TSTPS
PAIRW = 2 * OUT_DIM

_CHUNK = 128


def _sc_node_gather(node4, gid):
    info = plsc.get_sparse_core_info()
    nw = info.num_cores * info.num_subcores
    b_per_w = BS // nw

    mesh = plsc.VectorSubcoreMesh(core_axis_name="c", subcore_axis_name="s")

    @functools.partial(
        pl.kernel,
        mesh=mesh,
        compiler_params=pltpu.CompilerParams(use_tc_tiling_on_sc=True),
        out_type=jax.ShapeDtypeStruct((BS, TSTPS, OUT_DIM), jnp.float32),
        scratch_types=[
            pltpu.VMEM((b_per_w,), jnp.int32),
            pltpu.VMEM((8, TSTPS, OUT_DIM), jnp.float32),
            pltpu.VMEM((8, TSTPS, OUT_DIM), jnp.float32),
            pltpu.SemaphoreType.DMA,
            pltpu.SemaphoreType.DMA,
        ],
    )
    def k(node_hbm, gid_hbm, ng_out, gidx_v, buf0, buf1, sem, wsem):
        wid = lax.axis_index("s") * info.num_cores + lax.axis_index("c")
        base = wid * b_per_w
        pltpu.sync_copy(gid_hbm.at[wid], gidx_v)
        gv0 = gidx_v[pl.ds(0, 16)]
        gv1 = gidx_v[pl.ds(16, 16)]
        bufs = (buf0, buf1)
        wb = [None, None]
        for r in range(b_per_w // 8):
            bi = r % 2
            if wb[bi] is not None:
                wb[bi].wait()
            gs = []
            for i in range(8):
                idx = r * 8 + i
                g = (gv0 if idx < 16 else gv1)[idx % 16]
                gs.append(pltpu.async_copy(
                    node_hbm.at[base + idx, :, g, :], bufs[bi].at[i], sem))
            for c in gs:
                c.wait()
            wb[bi] = pltpu.async_copy(
                bufs[bi], ng_out.at[pl.ds(base + r * 8, 8)], wsem)
        wb[0].wait()
        wb[1].wait()

    return k(node4, gid)


def _sc_type_gather(table, gid):
    info = plsc.get_sparse_core_info()
    nw = info.num_cores * info.num_subcores
    g_per_w = BS // nw

    mesh = plsc.VectorSubcoreMesh(core_axis_name="c", subcore_axis_name="s")

    @functools.partial(
        pl.kernel,
        mesh=mesh,
        compiler_params=pltpu.CompilerParams(use_tc_tiling_on_sc=False),
        out_type=jax.ShapeDtypeStruct((nw, g_per_w, TPAD), jnp.float32),
        scratch_types=[
            pltpu.VMEM((g_per_w,), jnp.int32),
            pltpu.VMEM((g_per_w, TPAD), jnp.float32),
            pltpu.SemaphoreType.DMA,
        ],
    )
    def k(tbl_hbm, gid_hbm, gt_out, gidx_v, gt_v, sem):
        wid = lax.axis_index("s") * info.num_cores + lax.axis_index("c")
        pltpu.sync_copy(gid_hbm.at[wid], gidx_v)
        pltpu.async_copy(tbl_hbm.at[gidx_v], gt_v, sem).wait()
        pltpu.sync_copy(gt_v, gt_out.at[wid])

    return k(table, gid)


_BB = 16
_BLK = _BB * TSTPS


_BB = 32


def _tc_body(ng_ref, gt_ref, w1t_ref, w2t_ref, b_ref, out_ref):
    ng = ng_ref[...]
    gt = gt_ref[...][:, 0, :]
    acc = lax.dot_general(ng, w2t_ref[...], (((2,), (0,)), ((), ())),
                          preferred_element_type=jnp.float32)
    gtr = jnp.dot(gt, w1t_ref[...], preferred_element_type=jnp.float32)
    out_ref[...] = acc + gtr[:, None, :] + b_ref[...][None, :, :]


def _tc_fuse(ng3, gt, w1t, w2t, bvec):
    grid = (BS // _BB,)
    return pl.pallas_call(
        _tc_body,
        grid=grid,
        in_specs=[
            pl.BlockSpec((_BB, TSTPS, OUT_DIM), lambda i: (i, 0, 0)),
            pl.BlockSpec((_BB, 1, TPAD), lambda i: (i, 0, 0)),
            pl.BlockSpec((TPAD, OUT_DIM), lambda i: (0, 0)),
            pl.BlockSpec((OUT_DIM, OUT_DIM), lambda i: (0, 0)),
            pl.BlockSpec((1, OUT_DIM), lambda i: (0, 0)),
        ],
        out_specs=pl.BlockSpec((_BB, TSTPS, OUT_DIM), lambda i: (i, 0, 0)),
        out_shape=jax.ShapeDtypeStruct((BS, TSTPS, OUT_DIM), jnp.float32),
    )(ng3, gt, w1t, w2t, bvec)


def kernel(goal_id, goal_classnode_id, goal_node_id, node_repr, goal_type_table, W, b):
    del goal_classnode_id
    info = plsc.get_sparse_core_info()
    nw = info.num_cores * info.num_subcores

    gnid = goal_node_id.astype(jnp.int32).reshape(nw, BS // nw)
    ng3 = _sc_node_gather(node_repr, gnid)

    table_pad = jnp.pad(goal_type_table, ((0, 0), (0, TPAD - TYPE_DIM)))
    gt = _sc_type_gather(
        table_pad, goal_id.astype(jnp.int32).reshape(nw, BS // nw)
    ).reshape(BS, 1, TPAD)

    w1t = jnp.pad(W[:, :TYPE_DIM].T, ((0, TPAD - TYPE_DIM), (0, 0)))
    w2t = W[:, TYPE_DIM:].T
    return _tc_fuse(ng3, gt, w1t, w2t, b.reshape(1, OUT_DIM))
